# baseline (speedup 1.0000x reference)
import functools

import jax
import jax.numpy as jnp
from jax import lax
from jax.experimental import pallas as pl
from jax.experimental.pallas import tpu as pltpu
from jax.experimental.pallas import tpu_sc as plsc

VOCAB = 8192
DIM = 8192
BATCH = 16384

NUM_CORES = 2
NUM_SUBCORES = 16
NW = NUM_CORES * NUM_SUBCORES
B_SC = 10752
B_TC = BATCH - B_SC
BPW = B_SC // NW
CHUNK = 4
NBUF = 2
NCHUNKS = BPW // CHUNK
NT = NCHUNKS // NBUF


def _sc_gather(x, table):
    mesh = plsc.VectorSubcoreMesh(core_axis_name="c", subcore_axis_name="s")

    @functools.partial(
        pl.kernel,
        mesh=mesh,
        out_type=jax.ShapeDtypeStruct((B_SC, DIM), jnp.float32),
        scratch_types=[
            pltpu.VMEM((NCHUNKS, CHUNK), jnp.int32),
            pltpu.VMEM((NBUF, CHUNK, DIM), jnp.float32),
        ] + [pltpu.SemaphoreType.DMA] * NBUF,
    )
    def k(x_hbm, table_hbm, out_hbm, idx_v, bufs, *gsems):
        wid = lax.axis_index("s") * NUM_CORES + lax.axis_index("c")
        base = wid * BPW
        pltpu.sync_copy(x_hbm.at[wid], idx_v)

        def start_gather(g, b):
            pltpu.async_copy(
                table_hbm.at[idx_v.at[g]],
                bufs.at[b],
                gsems[b],
            )

        for b in range(NBUF):
            start_gather(b, b)

        def body(t, carry):
            for b in range(NBUF):
                g = t * NBUF + b
                pltpu.make_async_copy(
                    table_hbm.at[idx_v.at[0]],
                    bufs.at[b],
                    gsems[b],
                ).wait()
                pltpu.sync_copy(
                    bufs.at[b],
                    out_hbm.at[pl.ds(base + g * CHUNK, CHUNK)],
                )

                @pl.when(g + NBUF < NCHUNKS)
                def _():
                    start_gather(g + NBUF, b)

            return carry

        lax.fori_loop(0, NT, body, 0)

    return k(x, table)


def _tc_gather(idx, table):
    b = idx.shape[0]

    def body(idx_ref, table_ref, out_ref, sem):
        def issue(i, carry):
            r = idx_ref[i]
            pltpu.make_async_copy(
                table_ref.at[pl.ds(r, 1)], out_ref.at[pl.ds(i, 1)], sem
            ).start()
            return carry

        lax.fori_loop(0, b, issue, 0)
        pltpu.make_async_copy(table_ref.at[pl.ds(0, b)], out_ref, sem).wait()

    return pl.pallas_call(
        body,
        grid_spec=pltpu.PrefetchScalarGridSpec(
            num_scalar_prefetch=1,
            grid=(1,),
            in_specs=[pl.BlockSpec(memory_space=pl.ANY)],
            out_specs=pl.BlockSpec(memory_space=pl.ANY),
            scratch_shapes=[pltpu.SemaphoreType.DMA],
        ),
        out_shape=jax.ShapeDtypeStruct((b, DIM), jnp.float32),
    )(idx, table)


def kernel(x, table):
    xi = x.astype(jnp.int32)
    x3 = xi[:B_SC].reshape(NW, NCHUNKS, CHUNK)
    sc_out = _sc_gather(x3, table)
    tc_out = _tc_gather(xi[B_SC:], table)
    return jnp.concatenate([sc_out, tc_out], axis=0)


# speedup vs baseline: 14.9720x; 14.9720x over previous
import functools

import jax
import jax.numpy as jnp
from jax import lax
from jax.experimental import pallas as pl
from jax.experimental.pallas import tpu as pltpu
from jax.experimental.pallas import tpu_sc as plsc

VOCAB = 8192
DIM = 8192
BATCH = 16384

NUM_CORES = 2
NUM_SUBCORES = 16
NW = NUM_CORES * NUM_SUBCORES
BPW = BATCH // NW
CHUNK = 2
NBUF = 4
NCHUNKS = BPW // CHUNK
NT = NCHUNKS // NBUF


def _sc_gather(x, table):
    mesh = plsc.VectorSubcoreMesh(core_axis_name="c", subcore_axis_name="s")

    @functools.partial(
        pl.kernel,
        mesh=mesh,
        out_type=jax.ShapeDtypeStruct((BATCH, DIM), jnp.float32),
        scratch_types=[
            pltpu.VMEM((NCHUNKS, CHUNK), jnp.int32),
            pltpu.VMEM((NBUF, CHUNK, DIM), jnp.float32),
        ] + [pltpu.SemaphoreType.DMA] * (2 * NBUF),
    )
    def k(x_hbm, table_hbm, out_hbm, idx_v, bufs, *sems):
        gsems = sems[:NBUF]
        osems = sems[NBUF:]
        wid = lax.axis_index("s") * NUM_CORES + lax.axis_index("c")
        base = wid * BPW
        pltpu.sync_copy(x_hbm.at[wid], idx_v)

        def start_gather(g, b):
            pltpu.async_copy(table_hbm.at[idx_v.at[g]], bufs.at[b], gsems[b])

        def wait_gather(b):
            pltpu.make_async_copy(
                table_hbm.at[idx_v.at[0]], bufs.at[b], gsems[b]
            ).wait()

        def wait_out(b):
            pltpu.make_async_copy(
                bufs.at[b], out_hbm.at[pl.ds(base, CHUNK)], osems[b]
            ).wait()

        start_gather(0, 0)
        start_gather(1, 1)

        def trip(t, carry):
            for j in range(NBUF):
                g = t * NBUF + j
                wait_gather(j)
                pltpu.async_copy(
                    bufs.at[j],
                    out_hbm.at[pl.ds(base + g * CHUNK, CHUNK)],
                    osems[j],
                )
                bn = (j + 2) % NBUF
                if j < 2:
                    @pl.when(t > 0)
                    def _():
                        wait_out(bn)

                    start_gather(g + 2, bn)
                else:
                    wait_out(bn)

                    @pl.when(t < NT - 1)
                    def _():
                        start_gather(g + 2, bn)

            return carry

        lax.fori_loop(0, NT, trip, 0)
        wait_out(2)
        wait_out(3)

    return k(x, table)


def kernel(x, table):
    x3 = x.astype(jnp.int32).reshape(NW, NCHUNKS, CHUNK)
    return _sc_gather(x3, table)


# speedup vs baseline: 15.0194x; 1.0032x over previous
import functools

import jax
import jax.numpy as jnp
from jax import lax
from jax.experimental import pallas as pl
from jax.experimental.pallas import tpu as pltpu
from jax.experimental.pallas import tpu_sc as plsc

VOCAB = 8192
DIM = 8192
BATCH = 16384

NUM_CORES = 2
NUM_SUBCORES = 16
NW = NUM_CORES * NUM_SUBCORES
BPW = BATCH // NW
LANES = 16
NSUPER = BPW // LANES
CHUNK = 4
NBUF = 2
NCHUNKS = BPW // CHUNK


def _sc_gather(x, table):
    mesh = plsc.VectorSubcoreMesh(core_axis_name="c", subcore_axis_name="s")

    @functools.partial(
        pl.kernel,
        mesh=mesh,
        out_type=jax.ShapeDtypeStruct((BATCH, DIM), jnp.float32),
        scratch_types=[
            pltpu.VMEM((NSUPER, LANES), jnp.int32),
            pltpu.VMEM((NBUF, CHUNK, DIM), jnp.float32),
            pltpu.SemaphoreType.DMA,
            pltpu.SemaphoreType.DMA,
        ],
    )
    def k(x_hbm, table_hbm, out_hbm, idx_v, bufs, gsem0, gsem1):
        gsems = (gsem0, gsem1)
        wid = lax.axis_index("s") * NUM_CORES + lax.axis_index("c")
        base = wid * BPW
        pltpu.sync_copy(x_hbm.at[wid], idx_v)

        def start_rows(vec, lane0, b):
            for j in range(CHUNK):
                v = vec[lane0 + j]
                pltpu.async_copy(
                    table_hbm.at[pl.ds(v, 1)],
                    bufs.at[b].at[pl.ds(j, 1)],
                    gsems[b],
                )

        def wait_gather(b):
            pltpu.make_async_copy(
                table_hbm.at[pl.ds(0, CHUNK)], bufs.at[b], gsems[b]
            ).wait()

        vec0 = idx_v[0]
        start_rows(vec0, 0, 0)
        start_rows(vec0, CHUNK, 1)

        def sup(s, carry):
            vec_c = idx_v[s]
            vec_n = idx_v[jnp.minimum(s + 1, NSUPER - 1)]
            for j in range(4):
                c = 4 * s + j
                b = j % NBUF
                wait_gather(b)
                pltpu.sync_copy(
                    bufs.at[b],
                    out_hbm.at[pl.ds(base + c * CHUNK, CHUNK)],
                )
                tgt_vec = vec_c if j < 2 else vec_n
                lane0 = ((j + 2) % 4) * CHUNK

                @pl.when(c + 2 < NCHUNKS)
                def _():
                    start_rows(tgt_vec, lane0, b)

            return carry

        lax.fori_loop(0, NSUPER, sup, 0)

    return k(x, table)


def kernel(x, table):
    x3 = x.astype(jnp.int32).reshape(NW, NSUPER, LANES)
    return _sc_gather(x3, table)


# speedup vs baseline: 15.0844x; 1.0043x over previous
import functools

import jax
import jax.numpy as jnp
from jax import lax
from jax.experimental import pallas as pl
from jax.experimental.pallas import tpu as pltpu
from jax.experimental.pallas import tpu_sc as plsc

VOCAB = 8192
DIM = 8192
BATCH = 16384

NUM_CORES = 2
NUM_SUBCORES = 16
NW = NUM_CORES * NUM_SUBCORES
BPW = BATCH // NW
CHUNK = 4
NBUF = 2
NCHUNKS = BPW // CHUNK
NT = NCHUNKS // NBUF


def _sc_gather(x, table):
    mesh = plsc.VectorSubcoreMesh(core_axis_name="c", subcore_axis_name="s")

    @functools.partial(
        pl.kernel,
        mesh=mesh,
        out_type=jax.ShapeDtypeStruct((BATCH, DIM), jnp.float32),
        scratch_types=[
            pltpu.VMEM((NCHUNKS, CHUNK), jnp.int32),
            pltpu.VMEM((NBUF, CHUNK, DIM), jnp.float32),
        ] + [pltpu.SemaphoreType.DMA] * NBUF,
    )
    def k(x_hbm, table_hbm, out_hbm, idx_v, bufs, *gsems):
        wid = lax.axis_index("s") * NUM_CORES + lax.axis_index("c")
        base = wid * BPW
        pltpu.sync_copy(x_hbm.at[wid], idx_v)

        def start_gather(g, b):
            pltpu.async_copy(
                table_hbm.at[idx_v.at[g]],
                bufs.at[b],
                gsems[b],
            )

        for b in range(NBUF):
            start_gather(b, b)

        def body(t, carry):
            for b in range(NBUF):
                g = t * NBUF + b
                pltpu.make_async_copy(
                    table_hbm.at[idx_v.at[0]],
                    bufs.at[b],
                    gsems[b],
                ).wait()
                pltpu.sync_copy(
                    bufs.at[b],
                    out_hbm.at[pl.ds(base + g * CHUNK, CHUNK)],
                )

                @pl.when(g + NBUF < NCHUNKS)
                def _():
                    start_gather(g + NBUF, b)

            return carry

        lax.fori_loop(0, NT, body, 0)

    return k(x, table)


def kernel(x, table):
    x3 = x.astype(jnp.int32).reshape(NW, NCHUNKS, CHUNK)
    return _sc_gather(x3, table)
